# baseline (speedup 1.0000x reference)
import jax
import jax.numpy as jnp
from jax.experimental import pallas as pl


def _body(x_ref, y_ref, o_ref):
    yv = y_ref[...]
    xv = x_ref[...]
    nz = yv != 0.0
    inv = jnp.where(nz, 1.0 / jnp.where(nz, yv, 1.0), 0.0)
    o_ref[...] = inv * xv


def kernel(x, y):
    N = x.shape[0]
    COLS = 1024
    ROWS = N // COLS
    BR = 512
    x2 = x.reshape(ROWS, COLS)
    y2 = y.reshape(ROWS, COLS)
    out = pl.pallas_call(
        _body,
        grid=(ROWS // BR,),
        in_specs=[
            pl.BlockSpec((BR, COLS), lambda i: (i, 0)),
            pl.BlockSpec((BR, COLS), lambda i: (i, 0)),
        ],
        out_specs=pl.BlockSpec((BR, COLS), lambda i: (i, 0)),
        out_shape=jax.ShapeDtypeStruct((ROWS, COLS), jnp.float32),
    )(x2, y2)
    return out.reshape(N)


# speedup vs baseline: 4.0896x; 4.0896x over previous
import jax
import jax.numpy as jnp
from jax.experimental import pallas as pl


def _body(x_ref, y_ref, o_ref):
    yv = y_ref[...]
    xv = x_ref[...]
    nz = yv != 0.0
    inv = jnp.where(nz, 1.0 / jnp.where(nz, yv, 1.0), 0.0)
    o_ref[...] = inv * xv


def kernel(x, y):
    N = x.shape[0]
    BLK = 524288
    out = pl.pallas_call(
        _body,
        grid=(N // BLK,),
        in_specs=[
            pl.BlockSpec((BLK,), lambda i: (i,)),
            pl.BlockSpec((BLK,), lambda i: (i,)),
        ],
        out_specs=pl.BlockSpec((BLK,), lambda i: (i,)),
        out_shape=jax.ShapeDtypeStruct((N,), jnp.float32),
    )(x, y)
    return out


# speedup vs baseline: 4.1528x; 1.0155x over previous
import jax
import jax.numpy as jnp
from jax.experimental import pallas as pl


def _body(x_ref, y_ref, o_ref):
    yv = y_ref[...]
    xv = x_ref[...]
    nz = yv != 0.0
    inv = jnp.where(nz, 1.0 / jnp.where(nz, yv, 1.0), 0.0)
    o_ref[...] = inv * xv


def kernel(x, y):
    N = x.shape[0]
    BLK = 2097152
    out = pl.pallas_call(
        _body,
        grid=(N // BLK,),
        in_specs=[
            pl.BlockSpec((BLK,), lambda i: (i,)),
            pl.BlockSpec((BLK,), lambda i: (i,)),
        ],
        out_specs=pl.BlockSpec((BLK,), lambda i: (i,)),
        out_shape=jax.ShapeDtypeStruct((N,), jnp.float32),
    )(x, y)
    return out


# speedup vs baseline: 4.2057x; 1.0127x over previous
import jax
import jax.numpy as jnp
from jax.experimental import pallas as pl


def _body(x_ref, y_ref, o_ref):
    yv = y_ref[...]
    xv = x_ref[...]
    nz = yv != 0.0
    inv = jnp.where(nz, 1.0 / jnp.where(nz, yv, 1.0), 0.0)
    o_ref[...] = inv * xv


def kernel(x, y):
    N = x.shape[0]
    BLK = 1048576
    out = pl.pallas_call(
        _body,
        grid=(N // BLK,),
        in_specs=[
            pl.BlockSpec((BLK,), lambda i: (i,)),
            pl.BlockSpec((BLK,), lambda i: (i,)),
        ],
        out_specs=pl.BlockSpec((BLK,), lambda i: (i,)),
        out_shape=jax.ShapeDtypeStruct((N,), jnp.float32),
    )(x, y)
    return out
